# baseline (speedup 1.0000x reference)
import jax
import jax.numpy as jnp
from jax.experimental import pallas as pl
from jax.experimental.pallas import tpu as pltpu

B = 128
V = 100000
RB = 8
NSTEP = B // RB


def _body(a_ref, x_ref, out_ref):
    x = x_ref[...]
    m = jnp.max(x, axis=1, keepdims=True)
    s = jnp.sum(jnp.exp(x - m), axis=1, keepdims=True)
    cols = jax.lax.broadcasted_iota(jnp.int32, (1, V), 1)
    hit = cols == a_ref[...]
    g = jnp.sum(jnp.where(hit, x, 0.0), axis=1, keepdims=True)
    out_ref[...] = g - m - jnp.log(s)


@jax.jit
def kernel(prediction, action):
    action = action.astype(jnp.int32).reshape(B, 1)
    out = pl.pallas_call(
        _body,
        grid=(NSTEP,),
        in_specs=[
            pl.BlockSpec((RB, 1), lambda k: (k, 0)),
            pl.BlockSpec((RB, V), lambda k: (k, 0)),
        ],
        out_specs=pl.BlockSpec((RB, 1), lambda k: (k, 0)),
        out_shape=jax.ShapeDtypeStruct((B, 1), jnp.float32),
    )(action, prediction)
    return out.reshape(B)
